# baseline (speedup 1.0000x reference)
import functools
import math

import jax
import jax.numpy as jnp
from jax import lax
from jax.experimental import pallas as pl
from jax.experimental.pallas import tpu as pltpu
from jax.experimental.pallas import tpu_sc as plsc

B, T, D = 2, 2048, 1024
E, DFF, CAP = 8, 4096, 512
N = B * T

W = 528
NSLOT = E * W

NC, NS = 2, 16
NW = NC * NS
_MESH = dict(core_axis_name="c", subcore_axis_name="s")

_F32 = jnp.float32
_I32 = jnp.int32


def _excl_prefix16(m):
    lane = lax.broadcasted_iota(_I32, (E, N), 1) & 15
    x = m
    for s in (1, 2, 4, 8):
        shifted = jnp.concatenate(
            [jnp.zeros((E, s), _F32), x[:, : N - s]], axis=1)
        x = x + jnp.where(lane >= s, shifted, 0.0)
    return x - m


def _chunk_prefix(m):
    ri = lax.broadcasted_iota(_I32, (N, N // 16), 0)
    ci = lax.broadcasted_iota(_I32, (N, N // 16), 1)
    b16 = ((ri >> 4) == ci).astype(_F32)
    cnt = lax.dot_general(m, b16, (((1,), (0,)), ((), ())),
                          preferred_element_type=_F32,
                          precision=lax.Precision.HIGHEST)
    i16 = lax.broadcasted_iota(_I32, (N // 16, N // 16), 0)
    j16 = lax.broadcasted_iota(_I32, (N // 16, N // 16), 1)
    slt = (i16 < j16).astype(_F32)
    base = lax.dot_general(cnt, slt, (((1,), (0,)), ((), ())),
                           preferred_element_type=_F32,
                           precision=lax.Precision.HIGHEST)
    rj = lax.broadcasted_iota(_I32, (N // 16, N), 0)
    cj = lax.broadcasted_iota(_I32, (N // 16, N), 1)
    b16t = (rj == (cj >> 4)).astype(_F32)
    return lax.dot_general(base, b16t, (((1,), (0,)), ((), ())),
                           preferred_element_type=_F32,
                           precision=lax.Precision.HIGHEST)


def _router_body(wr_ref, x_ref, probs_ref, pos_ref):
    wr = wr_ref[...]
    x = x_ref[...]
    logits = lax.dot_general(
        wr, x, (((1,), (1,)), ((), ())),
        preferred_element_type=_F32,
    )
    mx = jnp.max(logits, axis=1, keepdims=True)
    p = jnp.exp(logits - mx)
    sm = jnp.sum(p, axis=1, keepdims=True)
    probs = p / sm
    probs_ref[...] = probs

    pbits = lax.bitcast_convert_type(probs, _I32)

    def step(_, carry):
        lo, hi = carry
        mid = lo + ((hi - lo) >> 1)
        cnt = jnp.sum((pbits >= mid).astype(_I32), axis=1, keepdims=True)
        ok = cnt >= CAP
        return jnp.where(ok, mid, lo), jnp.where(ok, hi, mid)

    lo0 = jnp.zeros((E, 1), _I32)
    hi0 = jnp.full((E, 1), 0x7F800000, _I32)
    tstar, _ = lax.fori_loop(0, 31, step, (lo0, hi0))

    m_gt = (pbits > tstar)
    m_eq = (pbits == tstar)
    mgt = m_gt.astype(_F32)
    meq = m_eq.astype(_F32)
    rank_gt = _chunk_prefix(mgt) + _excl_prefix16(mgt)
    tot_gt = jnp.sum(mgt, axis=1, keepdims=True)
    rank_eq = tot_gt + _chunk_prefix(meq) + _excl_prefix16(meq)

    pos = jnp.where(m_gt, rank_gt, jnp.where(m_eq, rank_eq, float(CAP)))
    pos = jnp.minimum(pos, float(CAP))
    eoff = lax.broadcasted_iota(_I32, (E, 1), 0).astype(_F32) * float(W)
    pos_ref[...] = (pos + eoff).astype(_I32)


def _router(x_flat, Wr):
    return pl.pallas_call(
        _router_body,
        out_shape=(
            jax.ShapeDtypeStruct((E, N), _F32),
            jax.ShapeDtypeStruct((E, N), _I32),
        ),
    )(Wr, x_flat)


def _select_body(pos_hbm, probs_hbm, idx_hbm, vals_hbm,
                 pos_v, p_v, ids_v, sem1, sem2):
    c = lax.axis_index("c")
    s = lax.axis_index("s")
    e = c * 4 + s

    @pl.when(s < 4)
    def _():
        pltpu.sync_copy(pos_hbm.at[e], pos_v)
        pltpu.sync_copy(probs_hbm.at[e], p_v)

        def fill(k, carry):
            ids_v[pl.ds(k * 16, 16)] = k * 16 + lax.iota(_I32, 16)
            return carry

        lax.fori_loop(0, N // 16, fill, 0, unroll=False)
        cp1 = pltpu.async_copy(ids_v, idx_hbm.at[pos_v], sem1)
        cp2 = pltpu.async_copy(p_v, vals_hbm.at[pos_v], sem2)
        cp1.wait()
        cp2.wait()


def _select(posg, probsT):
    return pl.kernel(
        _select_body,
        mesh=plsc.VectorSubcoreMesh(**_MESH),
        out_type=(
            jax.ShapeDtypeStruct((NSLOT,), _I32),
            jax.ShapeDtypeStruct((NSLOT,), _F32),
        ),
        scratch_types=[
            pltpu.VMEM((N,), _I32),
            pltpu.VMEM((N,), _F32),
            pltpu.VMEM((N,), _I32),
            pltpu.SemaphoreType.DMA,
            pltpu.SemaphoreType.DMA,
        ],
    )(posg, probsT)


_GCHUNK = 64


def _gather_body(x_hbm, idx_hbm, xe_hbm, idx_v, rows_v, sem):
    c = lax.axis_index("c")
    s = lax.axis_index("s")
    wid = s * NC + c
    rows_per_w = (E * CAP) // NW
    for k in range(rows_per_w // _GCHUNK):
        base = wid * rows_per_w + k * _GCHUNK
        pltpu.sync_copy(idx_hbm.at[pl.ds(base, _GCHUNK)], idx_v)
        pltpu.async_copy(x_hbm.at[idx_v], rows_v, sem).wait()
        pltpu.sync_copy(rows_v, xe_hbm.at[pl.ds(base, _GCHUNK)])


def _gather(x_flat, idx_flat):
    return pl.kernel(
        _gather_body,
        mesh=plsc.VectorSubcoreMesh(**_MESH),
        out_type=jax.ShapeDtypeStruct((E * CAP, D), _F32),
        scratch_types=[
            pltpu.VMEM((_GCHUNK,), _I32),
            pltpu.VMEM((_GCHUNK, D), _F32),
            pltpu.SemaphoreType.DMA,
        ],
    )(x_flat, idx_flat)


_DT = 1024
_NJ = DFF // _DT


def _ffn_body(xe_ref, w1_ref, b1_ref, w2_ref, b2_ref, vals_ref, y_ref, acc):
    j = pl.program_id(1)
    xb = xe_ref[0].astype(jnp.bfloat16)
    w1 = w1_ref[0].astype(jnp.bfloat16)
    h = lax.dot_general(xb, w1, (((1,), (1,)), ((), ())),
                        preferred_element_type=_F32)
    h = h + b1_ref[0]
    h = h * 0.5 * (1.0 + lax.erf(h * (1.0 / math.sqrt(2.0))))
    w2 = w2_ref[0].astype(jnp.bfloat16)
    contrib = lax.dot_general(h.astype(jnp.bfloat16), w2,
                              (((1,), (1,)), ((), ())),
                              preferred_element_type=_F32)

    @pl.when(j == 0)
    def _():
        acc[...] = contrib

    @pl.when(j > 0)
    def _():
        acc[...] += contrib

    @pl.when(j == _NJ - 1)
    def _():
        y = (acc[...] + b2_ref[0]) * vals_ref[0]
        y_ref[pl.ds(0, CAP), :] = y.astype(jnp.bfloat16)
        y_ref[pl.ds(CAP, W - CAP), :] = jnp.zeros((W - CAP, D), jnp.bfloat16)


def _ffn(xe, W1, b1, W2, b2, vals):
    b1r = b1.reshape(E, 1, DFF)
    b2r = b2.reshape(E, 1, D)
    vals3 = vals.reshape(E, CAP, 1)
    return pl.pallas_call(
        _ffn_body,
        grid=(E, _NJ),
        in_specs=[
            pl.BlockSpec((1, CAP, D), lambda e, j: (e, 0, 0)),
            pl.BlockSpec((1, _DT, D), lambda e, j: (e, j, 0)),
            pl.BlockSpec((1, 1, _DT), lambda e, j: (e, 0, j)),
            pl.BlockSpec((1, D, _DT), lambda e, j: (e, 0, j)),
            pl.BlockSpec((1, 1, D), lambda e, j: (e, 0, 0)),
            pl.BlockSpec((1, CAP, 1), lambda e, j: (e, 0, 0)),
        ],
        out_specs=pl.BlockSpec((W, D), lambda e, j: (e, 0)),
        out_shape=jax.ShapeDtypeStruct((NSLOT, D), jnp.bfloat16),
        scratch_shapes=[pltpu.VMEM((CAP, D), _F32)],
        compiler_params=pltpu.CompilerParams(
            dimension_semantics=("arbitrary", "arbitrary"),
        ),
    )(xe.reshape(E, CAP, D), W1, b1r, W2, b2r, vals3)


_TT = 512


def _combine_body(idx_ref, y_ref, out_ref, acc):
    t = pl.program_id(0)
    r = pl.program_id(1)
    idxrow = idx_ref[0]
    tio = lax.broadcasted_iota(_I32, (_TT, 1), 0) + t * _TT
    sel = (tio == idxrow).astype(jnp.bfloat16)
    contrib = lax.dot_general(sel, y_ref[...], (((1,), (0,)), ((), ())),
                              preferred_element_type=_F32)

    @pl.when(r == 0)
    def _():
        acc[...] = contrib

    @pl.when(r > 0)
    def _():
        acc[...] += contrib

    @pl.when(r == E - 1)
    def _():
        out_ref[...] = acc[...]


def _combine(idx2, yb):
    idx3 = idx2.reshape(E, 1, W)
    return pl.pallas_call(
        _combine_body,
        grid=(N // _TT, E),
        in_specs=[
            pl.BlockSpec((1, 1, W), lambda t, r: (r, 0, 0)),
            pl.BlockSpec((W, D), lambda t, r: (r, 0)),
        ],
        out_specs=pl.BlockSpec((_TT, D), lambda t, r: (t, 0)),
        out_shape=jax.ShapeDtypeStruct((N, D), _F32),
        scratch_shapes=[pltpu.VMEM((_TT, D), _F32)],
        compiler_params=pltpu.CompilerParams(
            dimension_semantics=("arbitrary", "arbitrary"),
        ),
    )(idx3, yb)


def kernel(x, Wr, W1, b1, W2, b2):
    x_flat = x.reshape(N, D)
    probsT, posg = _router(x_flat, Wr)
    idx2, vals2 = _select(posg, probsT)
    idx_flat = idx2.reshape(E, W)[:, :CAP].reshape(E * CAP)
    vals = vals2.reshape(E, W)[:, :CAP]
    xe = _gather(x_flat, idx_flat)
    yb = _ffn(xe, W1, b1, W2, b2, vals)
    out = _combine(idx2, yb)
    return out.reshape(B, T, D)
